# Initial kernel scaffold; baseline (speedup 1.0000x reference)
import jax
import jax.numpy as jnp
from jax.experimental import pallas as pl


def kernel(x, edge_index, edge_attr, poly1_index, poly1_val, poly2_index, poly2_val, all_poly_index, all_poly_val, all_loop_val, full_index, W_enc, b_enc, W0, b0, W1, b1, W2, b2, W_loop, W_conn, W_jk, b_jk, W_full, b_full, W_out, b_out):
    raise NotImplementedError("write your pallas kernel here")


# speedup vs baseline: 2.5117x; 2.5117x over previous
import functools
import jax
import jax.numpy as jnp
from jax import lax
from jax.experimental import pallas as pl
from jax.experimental.pallas import tpu as pltpu
from jax.experimental.pallas import tpu_sc as plsc

N = 10000
E = 160000
H = 128
EMB = 10
OUT = 64

NC = 2
NS = 16
NW = NC * NS
C = 128
N_ACC = 10112
TRASH = N
RPT = N_ACC // NS

F16 = H // 16

_sc_mesh = plsc.VectorSubcoreMesh(
    core_axis_name="c", subcore_axis_name="s", num_cores=NC, num_subcores=NS)


def _zero_rows(buf, nrows, ncols16):
    z = jnp.zeros((16,), jnp.float32)
    def zr(r, _):
        for k in range(ncols16):
            buf[r, pl.ds(k * 16, 16)] = z
        return 0
    lax.fori_loop(0, nrows, zr, 0, unroll=2)


CH_CONV = (E + NW * C - 1) // (NW * C)


def _conv_body(h_hbm, src_hbm, dst_hbm, val_hbm, out_hbm,
               src_v, dst_v, val_v, rows_v, acc_sh, sem):
    c = lax.axis_index("c")
    s = lax.axis_index("s")
    w = c * NS + s

    pltpu.sync_copy(src_hbm.at[w], src_v)
    pltpu.sync_copy(dst_hbm.at[w], dst_v)
    pltpu.sync_copy(val_hbm.at[w], val_v)

    _zero_rows(rows_v, C, F16)
    base = s * RPT
    for k in range(RPT // C):
        pltpu.sync_copy(rows_v, acc_sh.at[pl.ds(base + k * C, C)])
    rem = RPT % C
    if rem:
        pltpu.sync_copy(rows_v.at[pl.ds(0, rem)],
                        acc_sh.at[pl.ds(base + (RPT // C) * C, rem)])
    plsc.subcore_barrier()

    def chunk(j, _):
        pltpu.make_async_copy(h_hbm.at[src_v.at[j]], rows_v, sem).start()
        pltpu.make_async_copy(h_hbm.at[src_v.at[j]], rows_v, sem).wait()
        def scale(eb, _):
            v16 = val_v[pl.ds(j * C + eb * 16, 16)]
            for el in range(16):
                e = eb * 16 + el
                vb = lax.gather(
                    v16, jnp.full((16, 1), el, jnp.int32),
                    lax.GatherDimensionNumbers(
                        offset_dims=(), collapsed_slice_dims=(0,),
                        start_index_map=(0,)),
                    (1,), mode=lax.GatherScatterMode.PROMISE_IN_BOUNDS)
                for k in range(F16):
                    sl = pl.ds(k * 16, 16)
                    rows_v[e, sl] = rows_v[e, sl] * vb
            return 0
        lax.fori_loop(0, C // 16, scale, 0)
        pltpu.sync_copy(rows_v, acc_sh.at[dst_v.at[j]], add=True)
        return 0
    lax.fori_loop(0, CH_CONV, chunk, 0)

    plsc.subcore_barrier()
    pltpu.sync_copy(acc_sh.at[pl.ds(base, RPT)],
                    out_hbm.at[pl.ds(c * N_ACC + base, RPT)])


_conv_call = pl.kernel(
    _conv_body,
    out_type=jax.ShapeDtypeStruct((NC * N_ACC, H), jnp.float32),
    mesh=_sc_mesh,
    scratch_types=[
        pltpu.VMEM((CH_CONV, C), jnp.int32),
        pltpu.VMEM((CH_CONV, C), jnp.int32),
        pltpu.VMEM((CH_CONV * C,), jnp.float32),
        pltpu.VMEM((C, H), jnp.float32),
        pltpu.VMEM_SHARED((N_ACC, H), jnp.float32),
        pltpu.SemaphoreType.DMA,
    ],
)


CH_A = 80
CH_B = 40
EPT_B = E // NW
FULL_B = EPT_B // C
REM_B = EPT_B - FULL_B * C
G = 8


def _full_body(h_hbm, srcA_hbm, dstA_hbm, srcB_hbm, dstB_hbm, ea_hbm,
               out_hbm, srcA_v, dstA_v, srcB_v, dstB_v, rows_v, rows2_v,
               acc_sh, sem):
    c = lax.axis_index("c")
    s = lax.axis_index("s")
    w = c * NS + s

    _zero_rows(rows_v, C, F16)
    base = s * RPT
    for k in range(RPT // C):
        pltpu.sync_copy(rows_v, acc_sh.at[pl.ds(base + k * C, C)])
    rem = RPT % C
    if rem:
        pltpu.sync_copy(rows_v.at[pl.ds(0, rem)],
                        acc_sh.at[pl.ds(base + (RPT // C) * C, rem)])
    plsc.subcore_barrier()

    def groupA(jo, _):
        pltpu.sync_copy(srcA_hbm.at[w, pl.ds(jo * G, G)], srcA_v)
        pltpu.sync_copy(dstA_hbm.at[w, pl.ds(jo * G, G)], dstA_v)
        def chunkA(ji, _):
            pltpu.make_async_copy(h_hbm.at[srcA_v.at[ji]], rows_v, sem).start()
            pltpu.make_async_copy(h_hbm.at[srcA_v.at[ji]], rows_v, sem).wait()
            pltpu.sync_copy(rows_v, acc_sh.at[dstA_v.at[ji]], add=True)
            return 0
        lax.fori_loop(0, G, chunkA, 0)
        return 0
    lax.fori_loop(0, CH_A // G, groupA, 0)

    eb_base = w * EPT_B

    def groupB(jo, _):
        pltpu.sync_copy(srcB_hbm.at[w, pl.ds(jo * G, G)], srcB_v)
        pltpu.sync_copy(dstB_hbm.at[w, pl.ds(jo * G, G)], dstB_v)
        def chunkB(ji, _):
            j = jo * G + ji
            pltpu.make_async_copy(h_hbm.at[srcB_v.at[ji]], rows_v, sem).start()
            pltpu.make_async_copy(h_hbm.at[srcB_v.at[ji]], rows_v, sem).wait()
            pltpu.sync_copy(rows_v, acc_sh.at[dstB_v.at[ji]], add=True)

            @pl.when(j < FULL_B)
            def _():
                pltpu.sync_copy(ea_hbm.at[pl.ds(eb_base + j * C, C)], rows2_v)
                pltpu.sync_copy(rows2_v, acc_sh.at[dstB_v.at[ji]], add=True)
            return 0
        lax.fori_loop(0, G, chunkB, 0)
        return 0
    lax.fori_loop(0, CH_B // G, groupB, 0)

    pltpu.sync_copy(srcB_hbm.at[w, pl.ds(CH_B - G, G)], srcB_v)
    pltpu.sync_copy(dstB_hbm.at[w, pl.ds(CH_B - G, G)], dstB_v)
    pltpu.sync_copy(ea_hbm.at[pl.ds(eb_base + FULL_B * C, REM_B)],
                    rows2_v.at[pl.ds(0, REM_B)])
    pltpu.sync_copy(rows2_v, acc_sh.at[dstB_v.at[FULL_B - (CH_B - G)]],
                    add=True)

    plsc.subcore_barrier()
    pltpu.sync_copy(acc_sh.at[pl.ds(base, RPT)],
                    out_hbm.at[pl.ds(c * N_ACC + base, RPT)])


_full_call = pl.kernel(
    _full_body,
    out_type=jax.ShapeDtypeStruct((NC * N_ACC, H), jnp.float32),
    mesh=_sc_mesh,
    scratch_types=[
        pltpu.VMEM((G, C), jnp.int32),
        pltpu.VMEM((G, C), jnp.int32),
        pltpu.VMEM((G, C), jnp.int32),
        pltpu.VMEM((G, C), jnp.int32),
        pltpu.VMEM((C, H), jnp.float32),
        pltpu.VMEM((C, H), jnp.float32),
        pltpu.VMEM_SHARED((N_ACC, H), jnp.float32),
        pltpu.SemaphoreType.DMA,
    ],
)


CH_C = 40


def _poly_body(dstC_hbm, v_hbm, out_hbm, dstC_v, v_v, acc_sh, sem):
    c = lax.axis_index("c")
    s = lax.axis_index("s")
    w = c * NS + s

    _zero_rows(v_v, C, F16)
    base = s * RPT
    for k in range(RPT // C):
        pltpu.sync_copy(v_v, acc_sh.at[pl.ds(base + k * C, C)])
    rem = RPT % C
    if rem:
        pltpu.sync_copy(v_v.at[pl.ds(0, rem)],
                        acc_sh.at[pl.ds(base + (RPT // C) * C, rem)])
    plsc.subcore_barrier()

    def groupC(jo, _):
        pltpu.sync_copy(dstC_hbm.at[w, pl.ds(jo * G, G)], dstC_v)
        def chunkC(ji, _):
            j = jo * G + ji
            pltpu.sync_copy(v_hbm.at[pl.ds((w * CH_C + j) * C, C)], v_v)
            pltpu.sync_copy(v_v, acc_sh.at[dstC_v.at[ji]], add=True)
            return 0
        lax.fori_loop(0, G, chunkC, 0)
        return 0
    lax.fori_loop(0, CH_C // G, groupC, 0)

    plsc.subcore_barrier()
    pltpu.sync_copy(acc_sh.at[pl.ds(base, RPT)],
                    out_hbm.at[pl.ds(c * N_ACC + base, RPT)])


_poly_call = pl.kernel(
    _poly_body,
    out_type=jax.ShapeDtypeStruct((NC * N_ACC, H), jnp.float32),
    mesh=_sc_mesh,
    scratch_types=[
        pltpu.VMEM((G, C), jnp.int32),
        pltpu.VMEM((C, H), jnp.float32),
        pltpu.VMEM_SHARED((N_ACC, H), jnp.float32),
        pltpu.SemaphoreType.DMA,
    ],
)


_HI = lax.Precision.HIGHEST


def _bn_relu(y):
    m = jnp.mean(y, axis=0, keepdims=True)
    v = jnp.mean((y - m) * (y - m), axis=0, keepdims=True)
    return jnp.maximum((y - m) / jnp.sqrt(v + 1e-5), 0.0)


def _enc_body(x_ref, w_ref, b_ref, o_ref):
    y = jnp.dot(x_ref[...], w_ref[...], preferred_element_type=jnp.float32,
                precision=_HI) + b_ref[...]
    o_ref[...] = _bn_relu(y)


_enc_call = pl.pallas_call(
    _enc_body, out_shape=jax.ShapeDtypeStruct((N, H), jnp.float32))


def _post_conv_body(p_ref, h_ref, w_ref, b_ref, o_ref):
    agg = p_ref[pl.ds(0, N), :] + p_ref[pl.ds(N_ACC, N), :]
    y = jnp.dot(agg, w_ref[...], preferred_element_type=jnp.float32,
                precision=_HI) + b_ref[...]
    o_ref[...] = _bn_relu(y) + h_ref[...]


_post_conv_call = pl.pallas_call(
    _post_conv_body, out_shape=jax.ShapeDtypeStruct((N, H), jnp.float32))


def _jk1_body(h0_ref, h1_ref, lp_ref, wjk_ref, wl_ref, b_ref, o_ref):
    y = jnp.dot(h0_ref[...], wjk_ref[pl.ds(0, H), :],
                preferred_element_type=jnp.float32, precision=_HI)
    y += jnp.dot(h1_ref[...], wjk_ref[pl.ds(H, H), :],
                 preferred_element_type=jnp.float32, precision=_HI)
    wl = jnp.dot(wl_ref[...], wjk_ref[pl.ds(4 * H, H), :],
                 preferred_element_type=jnp.float32, precision=_HI)
    y += jnp.dot(lp_ref[...], wl, preferred_element_type=jnp.float32,
                 precision=_HI)
    o_ref[...] = y + b_ref[...]


_jk1_call = pl.pallas_call(
    _jk1_body, out_shape=jax.ShapeDtypeStruct((N, H), jnp.float32))


def _jk2_body(y_ref, h2_ref, h3_ref, wjk_ref, o_ref):
    y = y_ref[...]
    y += jnp.dot(h2_ref[...], wjk_ref[pl.ds(2 * H, H), :],
                 preferred_element_type=jnp.float32, precision=_HI)
    y += jnp.dot(h3_ref[...], wjk_ref[pl.ds(3 * H, H), :],
                 preferred_element_type=jnp.float32, precision=_HI)
    o_ref[...] = _bn_relu(y)


_jk2_call = pl.pallas_call(
    _jk2_body, out_shape=jax.ShapeDtypeStruct((N, H), jnp.float32))


def _final_body(p_ref, p16_ref, h_ref, wc_ref, wf_ref, bf_ref, wo_ref,
                bo_ref, o_ref):
    agg = p_ref[pl.ds(0, N), :] + p_ref[pl.ds(N_ACC, N), :]
    p16 = p16_ref[pl.ds(0, N), :] + p16_ref[pl.ds(N_ACC, N), :]
    agg += jnp.dot(p16, wc_ref[...], preferred_element_type=jnp.float32,
                   precision=_HI)
    y = jnp.dot(agg, wf_ref[...], preferred_element_type=jnp.float32,
                precision=_HI) + bf_ref[...]
    h5 = _bn_relu(y) + h_ref[...]
    o_ref[...] = jnp.dot(h5, wo_ref[...], preferred_element_type=jnp.float32,
                         precision=_HI) + bo_ref[...]


_final_call = pl.pallas_call(
    _final_body, out_shape=jax.ShapeDtypeStruct((N, OUT), jnp.float32))


def _slab(a, ch, fill):
    cap = NW * ch * C
    pad = cap - a.shape[0]
    if pad:
        a = jnp.concatenate([a, jnp.full((pad,), fill, a.dtype)])
    return a.reshape(NW, ch, C)


def _slab_pt(a, ch, fill):
    ept = a.shape[0] // NW
    a = a.reshape(NW, ept)
    pad = ch * C - ept
    a = jnp.concatenate([a, jnp.full((NW, pad), fill, a.dtype)], axis=1)
    return a.reshape(NW, ch, C)


def kernel(x, edge_index, edge_attr, poly1_index, poly1_val, poly2_index,
           poly2_val, all_poly_index, all_poly_val, all_loop_val, full_index,
           W_enc, b_enc, W0, b0, W1, b1, W2, b2, W_loop, W_conn, W_jk, b_jk,
           W_full, b_full, W_out, b_out):
    h = _enc_call(x, W_enc, b_enc.reshape(1, H))

    blocks = [
        (poly1_index, poly1_val, W0, b0),
        (poly2_index, poly2_val, W1, b1),
        (all_poly_index, all_poly_val[:, -1], W2, b2),
    ]
    nh = [h]
    for pidx, pval, W, b in blocks:
        src = _slab(pidx[0], CH_CONV, 0)
        dst = _slab(pidx[1], CH_CONV, TRASH)
        val = _slab(pval, CH_CONV, 0.0).reshape(NW, CH_CONV * C)
        part = _conv_call(h, src, dst, val)
        h = _post_conv_call(part, h, W, b.reshape(1, H))
        nh.append(h)

    y01 = _jk1_call(nh[0], nh[1], all_loop_val, W_jk, W_loop,
                    b_jk.reshape(1, H))
    h = _jk2_call(y01, nh[2], nh[3], W_jk)

    srcA = _slab(jnp.concatenate([full_index[0], all_poly_index[0]]), CH_A, 0)
    dstA = _slab(jnp.concatenate([full_index[1], all_poly_index[1]]), CH_A,
                 TRASH)
    srcB = _slab_pt(edge_index[0], CH_B, 0)
    dstB = _slab_pt(edge_index[1], CH_B, TRASH)
    dstC = _slab(all_poly_index[1], CH_C, TRASH)
    v128 = jnp.pad(all_poly_val, ((0, NW * CH_C * C - E), (0, H - EMB)))

    part16 = _poly_call(dstC, v128)
    part = _full_call(h, srcA, dstA, srcB, dstB, edge_attr)

    wc128 = jnp.pad(W_conn, ((0, H - EMB), (0, 0)))
    out = _final_call(part, part16, h, wc128, W_full, b_full.reshape(1, H),
                      W_out, b_out.reshape(1, OUT))
    return out


# speedup vs baseline: 2.9170x; 1.1614x over previous
import functools
import jax
import jax.numpy as jnp
from jax import lax
from jax.experimental import pallas as pl
from jax.experimental.pallas import tpu as pltpu
from jax.experimental.pallas import tpu_sc as plsc

N = 10000
E = 160000
H = 128
EMB = 10
OUT = 64

NC = 2
NS = 16
NW = NC * NS
C = 128
N_ACC = 10112
TRASH = N
RPT = N_ACC // NS

F16 = H // 16

_sc_mesh = plsc.VectorSubcoreMesh(
    core_axis_name="c", subcore_axis_name="s", num_cores=NC, num_subcores=NS)


def _zero_rows(buf, nrows, ncols16):
    z = jnp.zeros((16,), jnp.float32)
    def zr(r, _):
        for k in range(ncols16):
            buf[r, pl.ds(k * 16, 16)] = z
        return 0
    lax.fori_loop(0, nrows, zr, 0, unroll=2)


CH_CONV = (E + NW * C - 1) // (NW * C)


def _conv_body(h_hbm, src_hbm, dst_hbm, val_hbm, out_hbm,
               src_v, dst_v, val_v, r0, r1, acc_sh, sem0, sem1):
    c = lax.axis_index("c")
    s = lax.axis_index("s")
    w = c * NS + s

    pltpu.sync_copy(src_hbm.at[w], src_v)
    pltpu.sync_copy(dst_hbm.at[w], dst_v)
    pltpu.sync_copy(val_hbm.at[w], val_v)

    _zero_rows(r0, C, F16)
    base = s * RPT
    for k in range(RPT // C):
        pltpu.sync_copy(r0, acc_sh.at[pl.ds(base + k * C, C)])
    rem = RPT % C
    if rem:
        pltpu.sync_copy(r0.at[pl.ds(0, rem)],
                        acc_sh.at[pl.ds(base + (RPT // C) * C, rem)])
    plsc.subcore_barrier()

    def scale_scatter(j, rows):
        def scale(eb, _):
            v16 = val_v[pl.ds(j * C + eb * 16, 16)]
            for el in range(16):
                e = eb * 16 + el
                vb = lax.gather(
                    v16, jnp.full((16, 1), el, jnp.int32),
                    lax.GatherDimensionNumbers(
                        offset_dims=(), collapsed_slice_dims=(0,),
                        start_index_map=(0,)),
                    (1,), mode=lax.GatherScatterMode.PROMISE_IN_BOUNDS)
                for k in range(F16):
                    sl = pl.ds(k * 16, 16)
                    rows[e, sl] = rows[e, sl] * vb
            return 0
        lax.fori_loop(0, C // 16, scale, 0)
        pltpu.sync_copy(rows, acc_sh.at[dst_v.at[j]], add=True)

    pltpu.make_async_copy(h_hbm.at[src_v.at[0]], r0, sem0).start()

    def pair(j2, _):
        j = 2 * j2
        pltpu.make_async_copy(h_hbm.at[src_v.at[j + 1]], r1, sem1).start()
        pltpu.make_async_copy(h_hbm.at[src_v.at[j]], r0, sem0).wait()
        scale_scatter(j, r0)

        @pl.when(j + 2 < CH_CONV)
        def _():
            pltpu.make_async_copy(h_hbm.at[src_v.at[j + 2]], r0, sem0).start()
        pltpu.make_async_copy(h_hbm.at[src_v.at[j + 1]], r1, sem1).wait()
        scale_scatter(j + 1, r1)
        return 0
    lax.fori_loop(0, CH_CONV // 2, pair, 0)

    plsc.subcore_barrier()
    pltpu.sync_copy(acc_sh.at[pl.ds(base, RPT)],
                    out_hbm.at[pl.ds(c * N_ACC + base, RPT)])


_conv_call = pl.kernel(
    _conv_body,
    out_type=jax.ShapeDtypeStruct((NC * N_ACC, H), jnp.float32),
    mesh=_sc_mesh,
    scratch_types=[
        pltpu.VMEM((CH_CONV, C), jnp.int32),
        pltpu.VMEM((CH_CONV, C), jnp.int32),
        pltpu.VMEM((CH_CONV * C,), jnp.float32),
        pltpu.VMEM((C, H), jnp.float32),
        pltpu.VMEM((C, H), jnp.float32),
        pltpu.VMEM_SHARED((N_ACC, H), jnp.float32),
        pltpu.SemaphoreType.DMA,
        pltpu.SemaphoreType.DMA,
    ],
)


CH_A = 80
CH_B = 40
EPT_B = E // NW
FULL_B = EPT_B // C
REM_B = EPT_B - FULL_B * C
G = 8


def _full_body(h_hbm, srcA_hbm, dstA_hbm, srcB_hbm, dstB_hbm, ea_hbm,
               out_hbm, gsrc_v, gdst_v, srcB_v, dstB_v, r0, r1,
               acc_sh, sem0, sem1):
    c = lax.axis_index("c")
    s = lax.axis_index("s")
    w = c * NS + s

    pltpu.sync_copy(srcB_hbm.at[w], srcB_v)
    pltpu.sync_copy(dstB_hbm.at[w], dstB_v)

    _zero_rows(r0, C, F16)
    base = s * RPT
    for k in range(RPT // C):
        pltpu.sync_copy(r0, acc_sh.at[pl.ds(base + k * C, C)])
    rem = RPT % C
    if rem:
        pltpu.sync_copy(r0.at[pl.ds(0, rem)],
                        acc_sh.at[pl.ds(base + (RPT // C) * C, rem)])
    plsc.subcore_barrier()

    def groupA(jo, _):
        pltpu.sync_copy(srcA_hbm.at[w, pl.ds(jo * G, G)], gsrc_v)
        pltpu.sync_copy(dstA_hbm.at[w, pl.ds(jo * G, G)], gdst_v)
        pltpu.make_async_copy(h_hbm.at[gsrc_v.at[0]], r0, sem0).start()

        def pairA(i2, _):
            i = 2 * i2
            pltpu.make_async_copy(h_hbm.at[gsrc_v.at[i + 1]], r1, sem1).start()
            pltpu.make_async_copy(h_hbm.at[gsrc_v.at[i]], r0, sem0).wait()
            pltpu.sync_copy(r0, acc_sh.at[gdst_v.at[i]], add=True)

            @pl.when(i + 2 < G)
            def _():
                pltpu.make_async_copy(h_hbm.at[gsrc_v.at[i + 2]], r0,
                                      sem0).start()
            pltpu.make_async_copy(h_hbm.at[gsrc_v.at[i + 1]], r1, sem1).wait()
            pltpu.sync_copy(r1, acc_sh.at[gdst_v.at[i + 1]], add=True)
            return 0
        lax.fori_loop(0, G // 2, pairA, 0)
        return 0
    lax.fori_loop(0, CH_A // G, groupA, 0)

    eb_base = w * EPT_B
    pltpu.make_async_copy(h_hbm.at[srcB_v.at[0]], r0, sem0).start()

    def pairB(j2, _):
        j = 2 * j2
        pltpu.make_async_copy(h_hbm.at[srcB_v.at[j + 1]], r1, sem1).start()
        pltpu.make_async_copy(h_hbm.at[srcB_v.at[j]], r0, sem0).wait()
        pltpu.sync_copy(r0, acc_sh.at[dstB_v.at[j]], add=True)

        @pl.when(j < FULL_B)
        def _():
            pltpu.sync_copy(ea_hbm.at[pl.ds(eb_base + j * C, C)], r0)
            pltpu.sync_copy(r0, acc_sh.at[dstB_v.at[j]], add=True)

        @pl.when(j + 2 < CH_B)
        def _():
            pltpu.make_async_copy(h_hbm.at[srcB_v.at[j + 2]], r0, sem0).start()
        pltpu.make_async_copy(h_hbm.at[srcB_v.at[j + 1]], r1, sem1).wait()
        pltpu.sync_copy(r1, acc_sh.at[dstB_v.at[j + 1]], add=True)

        @pl.when(j + 1 < FULL_B)
        def _():
            pltpu.sync_copy(ea_hbm.at[pl.ds(eb_base + (j + 1) * C, C)], r1)
            pltpu.sync_copy(r1, acc_sh.at[dstB_v.at[j + 1]], add=True)
        return 0
    lax.fori_loop(0, CH_B // 2, pairB, 0)

    pltpu.sync_copy(ea_hbm.at[pl.ds(eb_base + FULL_B * C, REM_B)],
                    r0.at[pl.ds(0, REM_B)])
    pltpu.sync_copy(r0, acc_sh.at[dstB_v.at[FULL_B]], add=True)

    plsc.subcore_barrier()
    pltpu.sync_copy(acc_sh.at[pl.ds(base, RPT)],
                    out_hbm.at[pl.ds(c * N_ACC + base, RPT)])


_full_call = pl.kernel(
    _full_body,
    out_type=jax.ShapeDtypeStruct((NC * N_ACC, H), jnp.float32),
    mesh=_sc_mesh,
    scratch_types=[
        pltpu.VMEM((G, C), jnp.int32),
        pltpu.VMEM((G, C), jnp.int32),
        pltpu.VMEM((CH_B, C), jnp.int32),
        pltpu.VMEM((CH_B, C), jnp.int32),
        pltpu.VMEM((C, H), jnp.float32),
        pltpu.VMEM((C, H), jnp.float32),
        pltpu.VMEM_SHARED((N_ACC, H), jnp.float32),
        pltpu.SemaphoreType.DMA,
        pltpu.SemaphoreType.DMA,
    ],
)


CH_C = 40


def _poly_body(dstC_hbm, v_hbm, out_hbm, dstC_v, v0, v1, acc_sh, sem0, sem1):
    c = lax.axis_index("c")
    s = lax.axis_index("s")
    w = c * NS + s

    pltpu.sync_copy(dstC_hbm.at[w], dstC_v)

    _zero_rows(v0, C, F16)
    base = s * RPT
    for k in range(RPT // C):
        pltpu.sync_copy(v0, acc_sh.at[pl.ds(base + k * C, C)])
    rem = RPT % C
    if rem:
        pltpu.sync_copy(v0.at[pl.ds(0, rem)],
                        acc_sh.at[pl.ds(base + (RPT // C) * C, rem)])
    plsc.subcore_barrier()

    def vch(j):
        return v_hbm.at[pl.ds((w * CH_C + j) * C, C)]

    pltpu.make_async_copy(vch(0), v0, sem0).start()

    def pairC(j2, _):
        j = 2 * j2
        pltpu.make_async_copy(vch(j + 1), v1, sem1).start()
        pltpu.make_async_copy(vch(j), v0, sem0).wait()
        pltpu.sync_copy(v0, acc_sh.at[dstC_v.at[j]], add=True)

        @pl.when(j + 2 < CH_C)
        def _():
            pltpu.make_async_copy(vch(j + 2), v0, sem0).start()
        pltpu.make_async_copy(vch(j + 1), v1, sem1).wait()
        pltpu.sync_copy(v1, acc_sh.at[dstC_v.at[j + 1]], add=True)
        return 0
    lax.fori_loop(0, CH_C // 2, pairC, 0)

    plsc.subcore_barrier()
    pltpu.sync_copy(acc_sh.at[pl.ds(base, RPT)],
                    out_hbm.at[pl.ds(c * N_ACC + base, RPT)])


_poly_call = pl.kernel(
    _poly_body,
    out_type=jax.ShapeDtypeStruct((NC * N_ACC, H), jnp.float32),
    mesh=_sc_mesh,
    scratch_types=[
        pltpu.VMEM((CH_C, C), jnp.int32),
        pltpu.VMEM((C, H), jnp.float32),
        pltpu.VMEM((C, H), jnp.float32),
        pltpu.VMEM_SHARED((N_ACC, H), jnp.float32),
        pltpu.SemaphoreType.DMA,
        pltpu.SemaphoreType.DMA,
    ],
)


_HI = lax.Precision.HIGHEST


def _bn_relu(y):
    m = jnp.mean(y, axis=0, keepdims=True)
    v = jnp.mean((y - m) * (y - m), axis=0, keepdims=True)
    return jnp.maximum((y - m) / jnp.sqrt(v + 1e-5), 0.0)


def _enc_body(x_ref, w_ref, b_ref, o_ref):
    y = jnp.dot(x_ref[...], w_ref[...], preferred_element_type=jnp.float32,
                precision=_HI) + b_ref[...]
    o_ref[...] = _bn_relu(y)


_enc_call = pl.pallas_call(
    _enc_body, out_shape=jax.ShapeDtypeStruct((N, H), jnp.float32))


def _post_conv_body(p_ref, h_ref, w_ref, b_ref, o_ref):
    agg = p_ref[pl.ds(0, N), :] + p_ref[pl.ds(N_ACC, N), :]
    y = jnp.dot(agg, w_ref[...], preferred_element_type=jnp.float32,
                precision=_HI) + b_ref[...]
    o_ref[...] = _bn_relu(y) + h_ref[...]


_post_conv_call = pl.pallas_call(
    _post_conv_body, out_shape=jax.ShapeDtypeStruct((N, H), jnp.float32))


def _jk1_body(h0_ref, h1_ref, lp_ref, wjk_ref, wl_ref, b_ref, o_ref):
    y = jnp.dot(h0_ref[...], wjk_ref[pl.ds(0, H), :],
                preferred_element_type=jnp.float32, precision=_HI)
    y += jnp.dot(h1_ref[...], wjk_ref[pl.ds(H, H), :],
                 preferred_element_type=jnp.float32, precision=_HI)
    wl = jnp.dot(wl_ref[...], wjk_ref[pl.ds(4 * H, H), :],
                 preferred_element_type=jnp.float32, precision=_HI)
    y += jnp.dot(lp_ref[...], wl, preferred_element_type=jnp.float32,
                 precision=_HI)
    o_ref[...] = y + b_ref[...]


_jk1_call = pl.pallas_call(
    _jk1_body, out_shape=jax.ShapeDtypeStruct((N, H), jnp.float32))


def _jk2_body(y_ref, h2_ref, h3_ref, wjk_ref, o_ref):
    y = y_ref[...]
    y += jnp.dot(h2_ref[...], wjk_ref[pl.ds(2 * H, H), :],
                 preferred_element_type=jnp.float32, precision=_HI)
    y += jnp.dot(h3_ref[...], wjk_ref[pl.ds(3 * H, H), :],
                 preferred_element_type=jnp.float32, precision=_HI)
    o_ref[...] = _bn_relu(y)


_jk2_call = pl.pallas_call(
    _jk2_body, out_shape=jax.ShapeDtypeStruct((N, H), jnp.float32))


def _final_body(p_ref, p16_ref, h_ref, wc_ref, wf_ref, bf_ref, wo_ref,
                bo_ref, o_ref):
    agg = p_ref[pl.ds(0, N), :] + p_ref[pl.ds(N_ACC, N), :]
    p16 = p16_ref[pl.ds(0, N), :] + p16_ref[pl.ds(N_ACC, N), :]
    agg += jnp.dot(p16, wc_ref[...], preferred_element_type=jnp.float32,
                   precision=_HI)
    y = jnp.dot(agg, wf_ref[...], preferred_element_type=jnp.float32,
                precision=_HI) + bf_ref[...]
    h5 = _bn_relu(y) + h_ref[...]
    o_ref[...] = jnp.dot(h5, wo_ref[...], preferred_element_type=jnp.float32,
                         precision=_HI) + bo_ref[...]


_final_call = pl.pallas_call(
    _final_body, out_shape=jax.ShapeDtypeStruct((N, OUT), jnp.float32))


def _slab(a, ch, fill):
    cap = NW * ch * C
    pad = cap - a.shape[0]
    if pad:
        a = jnp.concatenate([a, jnp.full((pad,), fill, a.dtype)])
    return a.reshape(NW, ch, C)


def _slab_pt(a, ch, fill):
    ept = a.shape[0] // NW
    a = a.reshape(NW, ept)
    pad = ch * C - ept
    a = jnp.concatenate([a, jnp.full((NW, pad), fill, a.dtype)], axis=1)
    return a.reshape(NW, ch, C)


def kernel(x, edge_index, edge_attr, poly1_index, poly1_val, poly2_index,
           poly2_val, all_poly_index, all_poly_val, all_loop_val, full_index,
           W_enc, b_enc, W0, b0, W1, b1, W2, b2, W_loop, W_conn, W_jk, b_jk,
           W_full, b_full, W_out, b_out):
    h = _enc_call(x, W_enc, b_enc.reshape(1, H))

    blocks = [
        (poly1_index, poly1_val, W0, b0),
        (poly2_index, poly2_val, W1, b1),
        (all_poly_index, all_poly_val[:, -1], W2, b2),
    ]
    nh = [h]
    for pidx, pval, W, b in blocks:
        src = _slab(pidx[0], CH_CONV, 0)
        dst = _slab(pidx[1], CH_CONV, TRASH)
        val = _slab(pval, CH_CONV, 0.0).reshape(NW, CH_CONV * C)
        part = _conv_call(h, src, dst, val)
        h = _post_conv_call(part, h, W, b.reshape(1, H))
        nh.append(h)

    y01 = _jk1_call(nh[0], nh[1], all_loop_val, W_jk, W_loop,
                    b_jk.reshape(1, H))
    h = _jk2_call(y01, nh[2], nh[3], W_jk)

    srcA = _slab(jnp.concatenate([full_index[0], all_poly_index[0]]), CH_A, 0)
    dstA = _slab(jnp.concatenate([full_index[1], all_poly_index[1]]), CH_A,
                 TRASH)
    srcB = _slab_pt(edge_index[0], CH_B, 0)
    dstB = _slab_pt(edge_index[1], CH_B, TRASH)
    dstC = _slab(all_poly_index[1], CH_C, TRASH)
    v128 = jnp.pad(all_poly_val, ((0, NW * CH_C * C - E), (0, H - EMB)))

    part16 = _poly_call(dstC, v128)
    part = _full_call(h, srcA, dstA, srcB, dstB, edge_attr)

    wc128 = jnp.pad(W_conn, ((0, H - EMB), (0, 0)))
    out = _final_call(part, part16, h, wc128, W_full, b_full.reshape(1, H),
                      W_out, b_out.reshape(1, OUT))
    return out


# speedup vs baseline: 3.3701x; 1.1553x over previous
import functools
import jax
import jax.numpy as jnp
from jax import lax
from jax.experimental import pallas as pl
from jax.experimental.pallas import tpu as pltpu
from jax.experimental.pallas import tpu_sc as plsc

N = 10000
E = 160000
H = 128
EMB = 10
OUT = 64

NC = 2
NS = 16
NW = NC * NS
C = 128
N_ACC = 10112
TRASH = N
RPT = N_ACC // NS

F16 = H // 16

_sc_mesh = plsc.VectorSubcoreMesh(
    core_axis_name="c", subcore_axis_name="s", num_cores=NC, num_subcores=NS)


def _zero_rows(buf, nrows, ncols16):
    z = jnp.zeros((16,), jnp.float32)
    def zr(r, _):
        for k in range(ncols16):
            buf[r, pl.ds(k * 16, 16)] = z
        return 0
    lax.fori_loop(0, nrows, zr, 0, unroll=2)


CH_CONV = (E + NW * C - 1) // (NW * C)


def _conv_body(h_hbm, src_hbm, dst_hbm, val_hbm, out_hbm,
               src_v, dst_v, val_v, r0, r1, acc_sh, sem0, sem1):
    c = lax.axis_index("c")
    s = lax.axis_index("s")
    w = c * NS + s

    pltpu.sync_copy(src_hbm.at[w], src_v)
    pltpu.sync_copy(dst_hbm.at[w], dst_v)
    pltpu.sync_copy(val_hbm.at[w], val_v)

    _zero_rows(r0, C, F16)
    base = s * RPT
    for k in range(RPT // C):
        pltpu.sync_copy(r0, acc_sh.at[pl.ds(base + k * C, C)])
    rem = RPT % C
    if rem:
        pltpu.sync_copy(r0.at[pl.ds(0, rem)],
                        acc_sh.at[pl.ds(base + (RPT // C) * C, rem)])
    plsc.subcore_barrier()

    def scale_scatter(j, rows):
        def scale(eb, _):
            v16 = val_v[pl.ds(j * C + eb * 16, 16)]
            for el in range(16):
                e = eb * 16 + el
                vb = lax.gather(
                    v16, jnp.full((16, 1), el, jnp.int32),
                    lax.GatherDimensionNumbers(
                        offset_dims=(), collapsed_slice_dims=(0,),
                        start_index_map=(0,)),
                    (1,), mode=lax.GatherScatterMode.PROMISE_IN_BOUNDS)
                for k in range(F16):
                    sl = pl.ds(k * 16, 16)
                    rows[e, sl] = rows[e, sl] * vb
            return 0
        lax.fori_loop(0, C // 16, scale, 0)
        pltpu.sync_copy(rows, acc_sh.at[dst_v.at[j]], add=True)

    pltpu.make_async_copy(h_hbm.at[src_v.at[0]], r0, sem0).start()

    def pair(j2, _):
        j = 2 * j2
        pltpu.make_async_copy(h_hbm.at[src_v.at[j + 1]], r1, sem1).start()
        pltpu.make_async_copy(h_hbm.at[src_v.at[j]], r0, sem0).wait()
        scale_scatter(j, r0)

        @pl.when(j + 2 < CH_CONV)
        def _():
            pltpu.make_async_copy(h_hbm.at[src_v.at[j + 2]], r0, sem0).start()
        pltpu.make_async_copy(h_hbm.at[src_v.at[j + 1]], r1, sem1).wait()
        scale_scatter(j + 1, r1)
        return 0
    lax.fori_loop(0, CH_CONV // 2, pair, 0)

    plsc.subcore_barrier()
    pltpu.sync_copy(acc_sh.at[pl.ds(base, RPT)],
                    out_hbm.at[pl.ds(c * N_ACC + base, RPT)])


_conv_call = pl.kernel(
    _conv_body,
    out_type=jax.ShapeDtypeStruct((NC * N_ACC, H), jnp.float32),
    mesh=_sc_mesh,
    scratch_types=[
        pltpu.VMEM((CH_CONV, C), jnp.int32),
        pltpu.VMEM((CH_CONV, C), jnp.int32),
        pltpu.VMEM((CH_CONV * C,), jnp.float32),
        pltpu.VMEM((C, H), jnp.float32),
        pltpu.VMEM((C, H), jnp.float32),
        pltpu.VMEM_SHARED((N_ACC, H), jnp.float32),
        pltpu.SemaphoreType.DMA,
        pltpu.SemaphoreType.DMA,
    ],
)


CH_A = 80
CH_B = 40
EPT_B = E // NW
FULL_B = EPT_B // C
REM_B = EPT_B - FULL_B * C
G = 8


def _full_body(h_hbm, srcA_hbm, dstA_hbm, srcB_hbm, dstB_hbm, ea_hbm,
               out_hbm, gsrc_v, gdst_v, srcB_v, dstB_v, r0, r1,
               acc_sh, sem0, sem1):
    c = lax.axis_index("c")
    s = lax.axis_index("s")
    w = c * NS + s

    pltpu.sync_copy(srcB_hbm.at[w], srcB_v)
    pltpu.sync_copy(dstB_hbm.at[w], dstB_v)

    _zero_rows(r0, C, F16)
    base = s * RPT
    for k in range(RPT // C):
        pltpu.sync_copy(r0, acc_sh.at[pl.ds(base + k * C, C)])
    rem = RPT % C
    if rem:
        pltpu.sync_copy(r0.at[pl.ds(0, rem)],
                        acc_sh.at[pl.ds(base + (RPT // C) * C, rem)])
    plsc.subcore_barrier()

    def groupA(jo, _):
        pltpu.sync_copy(srcA_hbm.at[w, pl.ds(jo * G, G)], gsrc_v)
        pltpu.sync_copy(dstA_hbm.at[w, pl.ds(jo * G, G)], gdst_v)
        pltpu.make_async_copy(h_hbm.at[gsrc_v.at[0]], r0, sem0).start()

        def pairA(i2, _):
            i = 2 * i2
            pltpu.make_async_copy(h_hbm.at[gsrc_v.at[i + 1]], r1, sem1).start()
            pltpu.make_async_copy(h_hbm.at[gsrc_v.at[i]], r0, sem0).wait()
            pltpu.sync_copy(r0, acc_sh.at[gdst_v.at[i]], add=True)

            @pl.when(i + 2 < G)
            def _():
                pltpu.make_async_copy(h_hbm.at[gsrc_v.at[i + 2]], r0,
                                      sem0).start()
            pltpu.make_async_copy(h_hbm.at[gsrc_v.at[i + 1]], r1, sem1).wait()
            pltpu.sync_copy(r1, acc_sh.at[gdst_v.at[i + 1]], add=True)
            return 0
        lax.fori_loop(0, G // 2, pairA, 0)
        return 0
    lax.fori_loop(0, CH_A // G, groupA, 0)

    eb_base = w * EPT_B
    pltpu.make_async_copy(h_hbm.at[srcB_v.at[0]], r0, sem0).start()

    def pairB(j2, _):
        j = 2 * j2
        pltpu.make_async_copy(h_hbm.at[srcB_v.at[j + 1]], r1, sem1).start()
        pltpu.make_async_copy(h_hbm.at[srcB_v.at[j]], r0, sem0).wait()
        pltpu.sync_copy(r0, acc_sh.at[dstB_v.at[j]], add=True)

        @pl.when(j < FULL_B)
        def _():
            pltpu.sync_copy(ea_hbm.at[pl.ds(eb_base + j * C, C)], r0)
            pltpu.sync_copy(r0, acc_sh.at[dstB_v.at[j]], add=True)

        @pl.when(j + 2 < CH_B)
        def _():
            pltpu.make_async_copy(h_hbm.at[srcB_v.at[j + 2]], r0, sem0).start()
        pltpu.make_async_copy(h_hbm.at[srcB_v.at[j + 1]], r1, sem1).wait()
        pltpu.sync_copy(r1, acc_sh.at[dstB_v.at[j + 1]], add=True)

        @pl.when(j + 1 < FULL_B)
        def _():
            pltpu.sync_copy(ea_hbm.at[pl.ds(eb_base + (j + 1) * C, C)], r1)
            pltpu.sync_copy(r1, acc_sh.at[dstB_v.at[j + 1]], add=True)
        return 0
    lax.fori_loop(0, CH_B // 2, pairB, 0)

    pltpu.sync_copy(ea_hbm.at[pl.ds(eb_base + FULL_B * C, REM_B)],
                    r0.at[pl.ds(0, REM_B)])
    pltpu.sync_copy(r0, acc_sh.at[dstB_v.at[FULL_B]], add=True)

    plsc.subcore_barrier()
    pltpu.sync_copy(acc_sh.at[pl.ds(base, RPT)],
                    out_hbm.at[pl.ds(c * N_ACC + base, RPT)])


_full_call = pl.kernel(
    _full_body,
    out_type=jax.ShapeDtypeStruct((NC * N_ACC, H), jnp.float32),
    mesh=_sc_mesh,
    scratch_types=[
        pltpu.VMEM((G, C), jnp.int32),
        pltpu.VMEM((G, C), jnp.int32),
        pltpu.VMEM((CH_B, C), jnp.int32),
        pltpu.VMEM((CH_B, C), jnp.int32),
        pltpu.VMEM((C, H), jnp.float32),
        pltpu.VMEM((C, H), jnp.float32),
        pltpu.VMEM_SHARED((N_ACC, H), jnp.float32),
        pltpu.SemaphoreType.DMA,
        pltpu.SemaphoreType.DMA,
    ],
)


CH_C = 40


def _poly_body(dstC_hbm, v_hbm, out_hbm, dstC_v, v0, v1, acc_sh, sem0, sem1):
    c = lax.axis_index("c")
    s = lax.axis_index("s")
    w = c * NS + s

    pltpu.sync_copy(dstC_hbm.at[w], dstC_v)

    _zero_rows(v0, C, F16)
    base = s * RPT
    for k in range(RPT // C):
        pltpu.sync_copy(v0, acc_sh.at[pl.ds(base + k * C, C)])
    rem = RPT % C
    if rem:
        pltpu.sync_copy(v0.at[pl.ds(0, rem)],
                        acc_sh.at[pl.ds(base + (RPT // C) * C, rem)])
    plsc.subcore_barrier()

    def vch(j):
        return v_hbm.at[pl.ds((w * CH_C + j) * C, C)]

    pltpu.make_async_copy(vch(0), v0, sem0).start()

    def pairC(j2, _):
        j = 2 * j2
        pltpu.make_async_copy(vch(j + 1), v1, sem1).start()
        pltpu.make_async_copy(vch(j), v0, sem0).wait()
        pltpu.sync_copy(v0, acc_sh.at[dstC_v.at[j]], add=True)

        @pl.when(j + 2 < CH_C)
        def _():
            pltpu.make_async_copy(vch(j + 2), v0, sem0).start()
        pltpu.make_async_copy(vch(j + 1), v1, sem1).wait()
        pltpu.sync_copy(v1, acc_sh.at[dstC_v.at[j + 1]], add=True)
        return 0
    lax.fori_loop(0, CH_C // 2, pairC, 0)

    plsc.subcore_barrier()
    pltpu.sync_copy(acc_sh.at[pl.ds(base, RPT)],
                    out_hbm.at[pl.ds(c * N_ACC + base, RPT)])


_poly_call = pl.kernel(
    _poly_body,
    out_type=jax.ShapeDtypeStruct((NC * N_ACC, H), jnp.float32),
    mesh=_sc_mesh,
    scratch_types=[
        pltpu.VMEM((CH_C, C), jnp.int32),
        pltpu.VMEM((C, H), jnp.float32),
        pltpu.VMEM((C, H), jnp.float32),
        pltpu.VMEM_SHARED((N_ACC, H), jnp.float32),
        pltpu.SemaphoreType.DMA,
        pltpu.SemaphoreType.DMA,
    ],
)


_HI = lax.Precision.HIGHEST


def _bn_relu(y):
    m = jnp.mean(y, axis=0, keepdims=True)
    v = jnp.mean((y - m) * (y - m), axis=0, keepdims=True)
    return jnp.maximum((y - m) / jnp.sqrt(v + 1e-5), 0.0)


def _enc_body(x_ref, w_ref, b_ref, o_ref):
    y = jnp.dot(x_ref[...], w_ref[...], preferred_element_type=jnp.float32,
                precision=_HI) + b_ref[...]
    o_ref[...] = _bn_relu(y)


_enc_call = pl.pallas_call(
    _enc_body, out_shape=jax.ShapeDtypeStruct((N, H), jnp.float32))


def _post_conv_body(p_ref, h_ref, w_ref, b_ref, o_ref):
    agg = p_ref[pl.ds(0, N), :] + p_ref[pl.ds(N_ACC, N), :]
    y = jnp.dot(agg, w_ref[...], preferred_element_type=jnp.float32,
                precision=_HI) + b_ref[...]
    o_ref[...] = _bn_relu(y) + h_ref[...]


_post_conv_call = pl.pallas_call(
    _post_conv_body, out_shape=jax.ShapeDtypeStruct((N, H), jnp.float32))


def _jk1_body(h0_ref, h1_ref, lp_ref, wjk_ref, wl_ref, b_ref, o_ref):
    y = jnp.dot(h0_ref[...], wjk_ref[pl.ds(0, H), :],
                preferred_element_type=jnp.float32, precision=_HI)
    y += jnp.dot(h1_ref[...], wjk_ref[pl.ds(H, H), :],
                 preferred_element_type=jnp.float32, precision=_HI)
    wl = jnp.dot(wl_ref[...], wjk_ref[pl.ds(4 * H, H), :],
                 preferred_element_type=jnp.float32, precision=_HI)
    y += jnp.dot(lp_ref[...], wl, preferred_element_type=jnp.float32,
                 precision=_HI)
    o_ref[...] = y + b_ref[...]


_jk1_call = pl.pallas_call(
    _jk1_body, out_shape=jax.ShapeDtypeStruct((N, H), jnp.float32))


def _jk2_body(y_ref, h2_ref, h3_ref, wjk_ref, o_ref):
    y = y_ref[...]
    y += jnp.dot(h2_ref[...], wjk_ref[pl.ds(2 * H, H), :],
                 preferred_element_type=jnp.float32, precision=_HI)
    y += jnp.dot(h3_ref[...], wjk_ref[pl.ds(3 * H, H), :],
                 preferred_element_type=jnp.float32, precision=_HI)
    o_ref[...] = _bn_relu(y)


_jk2_call = pl.pallas_call(
    _jk2_body, out_shape=jax.ShapeDtypeStruct((N, H), jnp.float32))


def _final_body(p_ref, p16_ref, h_ref, wc_ref, wf_ref, bf_ref, wo_ref,
                bo_ref, o_ref):
    agg = p_ref[pl.ds(0, N), :] + p_ref[pl.ds(N_ACC, N), :]
    p16 = p16_ref[pl.ds(0, N), :] + p16_ref[pl.ds(N_ACC, N), :]
    agg += jnp.dot(p16, wc_ref[...], preferred_element_type=jnp.float32,
                   precision=_HI)
    y = jnp.dot(agg, wf_ref[...], preferred_element_type=jnp.float32,
                precision=_HI) + bf_ref[...]
    h5 = _bn_relu(y) + h_ref[...]
    o_ref[...] = jnp.dot(h5, wo_ref[...], preferred_element_type=jnp.float32,
                         precision=_HI) + bo_ref[...]


_final_call = pl.pallas_call(
    _final_body, out_shape=jax.ShapeDtypeStruct((N, OUT), jnp.float32))


def _slab(a, ch, fill):
    cap = NW * ch * C
    pad = cap - a.shape[0]
    if pad:
        a = jnp.concatenate([a, jnp.full((pad,), fill, a.dtype)])
    return a.reshape(NW, ch, C)


def _slab_il(a, ch, fill):
    cap = NW * ch * C
    pad = cap - a.shape[0]
    if pad:
        a = jnp.concatenate([a, jnp.full((pad,), fill, a.dtype)])
    return a.reshape(ch, NW, C).swapaxes(0, 1)


def _slab_pt(a, ch, fill):
    ept = a.shape[0] // NW
    a = a.reshape(NW, ept)
    pad = ch * C - ept
    a = jnp.concatenate([a, jnp.full((NW, pad), fill, a.dtype)], axis=1)
    return a.reshape(NW, ch, C)


def kernel(x, edge_index, edge_attr, poly1_index, poly1_val, poly2_index,
           poly2_val, all_poly_index, all_poly_val, all_loop_val, full_index,
           W_enc, b_enc, W0, b0, W1, b1, W2, b2, W_loop, W_conn, W_jk, b_jk,
           W_full, b_full, W_out, b_out):
    h = _enc_call(x, W_enc, b_enc.reshape(1, H))

    blocks = [
        (poly1_index, poly1_val, W0, b0),
        (poly2_index, poly2_val, W1, b1),
        (all_poly_index, all_poly_val[:, -1], W2, b2),
    ]
    nh = [h]
    for pidx, pval, W, b in blocks:
        src = _slab_il(pidx[0], CH_CONV, 0)
        dst = _slab_il(pidx[1], CH_CONV, TRASH)
        val = _slab_il(pval, CH_CONV, 0.0).reshape(NW, CH_CONV * C)
        part = _conv_call(h, src, dst, val)
        h = _post_conv_call(part, h, W, b.reshape(1, H))
        nh.append(h)

    y01 = _jk1_call(nh[0], nh[1], all_loop_val, W_jk, W_loop,
                    b_jk.reshape(1, H))
    h = _jk2_call(y01, nh[2], nh[3], W_jk)

    srcA = _slab_il(jnp.concatenate([full_index[0], all_poly_index[0]]),
                    CH_A, 0)
    dstA = _slab_il(jnp.concatenate([full_index[1], all_poly_index[1]]),
                    CH_A, TRASH)
    srcB = _slab_pt(edge_index[0], CH_B, 0)
    dstB = _slab_pt(edge_index[1], CH_B, TRASH)
    dstC = _slab(all_poly_index[1], CH_C, TRASH)
    v128 = jnp.pad(all_poly_val, ((0, NW * CH_C * C - E), (0, H - EMB)))

    part16 = _poly_call(dstC, v128)
    part = _full_call(h, srcA, dstA, srcB, dstB, edge_attr)

    wc128 = jnp.pad(W_conn, ((0, H - EMB), (0, 0)))
    out = _final_call(part, part16, h, wc128, W_full, b_full.reshape(1, H),
                      W_out, b_out.reshape(1, OUT))
    return out


# speedup vs baseline: 4.1638x; 1.2355x over previous
import functools
import jax
import jax.numpy as jnp
from jax import lax
from jax.experimental import pallas as pl
from jax.experimental.pallas import tpu as pltpu
from jax.experimental.pallas import tpu_sc as plsc

N = 10000
E = 160000
H = 128
EMB = 10
OUT = 64

NC = 2
NS = 16
NW = NC * NS
C = 128
N_ACC = 10112
TRASH = N
RPT = N_ACC // NS

F16 = H // 16

_sc_mesh = plsc.VectorSubcoreMesh(
    core_axis_name="c", subcore_axis_name="s", num_cores=NC, num_subcores=NS)


def _zero_rows(buf, nrows, ncols16):
    z = jnp.zeros((16,), jnp.float32)
    def zr(r, _):
        for k in range(ncols16):
            buf[r, pl.ds(k * 16, 16)] = z
        return 0
    lax.fori_loop(0, nrows, zr, 0, unroll=2)


CC = 80
CH_CONV = 63


def _conv_body(h_hbm, src_hbm, dst_hbm, val_hbm, out_hbm,
               src_v, dst_v, val_v, r0, r1, r2, acc_sh,
               sg0, sg1, sg2, ss0, ss1, ss2):
    c = lax.axis_index("c")
    s = lax.axis_index("s")
    w = c * NS + s

    pltpu.sync_copy(src_hbm.at[w], src_v)
    pltpu.sync_copy(dst_hbm.at[w], dst_v)
    pltpu.sync_copy(val_hbm.at[w], val_v)

    _zero_rows(r0, CC, F16)
    base = s * RPT
    for k in range(RPT // CC):
        pltpu.sync_copy(r0, acc_sh.at[pl.ds(base + k * CC, CC)])
    rem = RPT % CC
    if rem:
        pltpu.sync_copy(r0.at[pl.ds(0, rem)],
                        acc_sh.at[pl.ds(base + (RPT // CC) * CC, rem)])
    plsc.subcore_barrier()

    def scale(j, rows):
        def sc16(eb, _):
            v16 = val_v[pl.ds(j * CC + eb * 16, 16)]
            for el in range(16):
                e = eb * 16 + el
                vb = lax.gather(
                    v16, jnp.full((16, 1), el, jnp.int32),
                    lax.GatherDimensionNumbers(
                        offset_dims=(), collapsed_slice_dims=(0,),
                        start_index_map=(0,)),
                    (1,), mode=lax.GatherScatterMode.PROMISE_IN_BOUNDS)
                for k in range(F16):
                    sl = pl.ds(k * 16, 16)
                    rows[e, sl] = rows[e, sl] * vb
            return 0
        lax.fori_loop(0, CC // 16, sc16, 0)

    def gcp(j, buf, sem):
        return pltpu.make_async_copy(
            h_hbm.at[src_v.at[pl.ds(j * CC, CC)]], buf, sem)

    def scp(j, buf, sem):
        return pltpu.make_async_copy(
            buf, acc_sh.at[dst_v.at[pl.ds(j * CC, CC)]], sem)

    gcp(0, r0, sg0).start()
    gcp(1, r1, sg1).start()

    def triple(j3, _):
        j = 3 * j3
        gcp(j, r0, sg0).wait()
        scale(j, r0)
        scp(j, r0, ss0).start(add=True)

        @pl.when(j3 > 0)
        def _():
            scp(j - 1, r2, ss2).wait()
        gcp(j + 2, r2, sg2).start()

        gcp(j + 1, r1, sg1).wait()
        scale(j + 1, r1)
        scp(j + 1, r1, ss1).start(add=True)
        scp(j, r0, ss0).wait()

        @pl.when(j + 3 < CH_CONV)
        def _():
            gcp(j + 3, r0, sg0).start()

        gcp(j + 2, r2, sg2).wait()
        scale(j + 2, r2)
        scp(j + 2, r2, ss2).start(add=True)
        scp(j + 1, r1, ss1).wait()

        @pl.when(j + 4 < CH_CONV)
        def _():
            gcp(j + 4, r1, sg1).start()
        return 0
    lax.fori_loop(0, CH_CONV // 3, triple, 0)
    scp(CH_CONV - 1, r2, ss2).wait()

    plsc.subcore_barrier()
    pltpu.sync_copy(acc_sh.at[pl.ds(base, RPT)],
                    out_hbm.at[pl.ds(c * N_ACC + base, RPT)])


_conv_call = pl.kernel(
    _conv_body,
    out_type=jax.ShapeDtypeStruct((NC * N_ACC, H), jnp.float32),
    mesh=_sc_mesh,
    scratch_types=[
        pltpu.VMEM((CH_CONV * CC,), jnp.int32),
        pltpu.VMEM((CH_CONV * CC,), jnp.int32),
        pltpu.VMEM((CH_CONV * CC,), jnp.float32),
        pltpu.VMEM((CC, H), jnp.float32),
        pltpu.VMEM((CC, H), jnp.float32),
        pltpu.VMEM((CC, H), jnp.float32),
        pltpu.VMEM_SHARED((N_ACC, H), jnp.float32),
        pltpu.SemaphoreType.DMA,
        pltpu.SemaphoreType.DMA,
        pltpu.SemaphoreType.DMA,
        pltpu.SemaphoreType.DMA,
        pltpu.SemaphoreType.DMA,
        pltpu.SemaphoreType.DMA,
    ],
)


CH_A = 80
CH_B = 40
EPT_B = E // NW
FULL_B = EPT_B // C
REM_B = EPT_B - FULL_B * C
G = 8


def _full_body(h_hbm, srcA_hbm, dstA_hbm, srcB_hbm, dstB_hbm, ea_hbm,
               out_hbm, gsrc_v, gdst_v, srcB_v, dstB_v, r0, r1,
               acc_sh, sem0, sem1):
    c = lax.axis_index("c")
    s = lax.axis_index("s")
    w = c * NS + s

    pltpu.sync_copy(srcB_hbm.at[w], srcB_v)
    pltpu.sync_copy(dstB_hbm.at[w], dstB_v)

    _zero_rows(r0, C, F16)
    base = s * RPT
    for k in range(RPT // C):
        pltpu.sync_copy(r0, acc_sh.at[pl.ds(base + k * C, C)])
    rem = RPT % C
    if rem:
        pltpu.sync_copy(r0.at[pl.ds(0, rem)],
                        acc_sh.at[pl.ds(base + (RPT // C) * C, rem)])
    plsc.subcore_barrier()

    def groupA(jo, _):
        pltpu.sync_copy(srcA_hbm.at[w, pl.ds(jo * G, G)], gsrc_v)
        pltpu.sync_copy(dstA_hbm.at[w, pl.ds(jo * G, G)], gdst_v)
        pltpu.make_async_copy(h_hbm.at[gsrc_v.at[0]], r0, sem0).start()

        def pairA(i2, _):
            i = 2 * i2
            pltpu.make_async_copy(h_hbm.at[gsrc_v.at[i + 1]], r1, sem1).start()
            pltpu.make_async_copy(h_hbm.at[gsrc_v.at[i]], r0, sem0).wait()
            pltpu.sync_copy(r0, acc_sh.at[gdst_v.at[i]], add=True)

            @pl.when(i + 2 < G)
            def _():
                pltpu.make_async_copy(h_hbm.at[gsrc_v.at[i + 2]], r0,
                                      sem0).start()
            pltpu.make_async_copy(h_hbm.at[gsrc_v.at[i + 1]], r1, sem1).wait()
            pltpu.sync_copy(r1, acc_sh.at[gdst_v.at[i + 1]], add=True)
            return 0
        lax.fori_loop(0, G // 2, pairA, 0)
        return 0
    lax.fori_loop(0, CH_A // G, groupA, 0)

    eb_base = w * EPT_B
    pltpu.make_async_copy(h_hbm.at[srcB_v.at[0]], r0, sem0).start()

    def pairB(j2, _):
        j = 2 * j2
        pltpu.make_async_copy(h_hbm.at[srcB_v.at[j + 1]], r1, sem1).start()
        pltpu.make_async_copy(h_hbm.at[srcB_v.at[j]], r0, sem0).wait()
        pltpu.sync_copy(r0, acc_sh.at[dstB_v.at[j]], add=True)

        @pl.when(j < FULL_B)
        def _():
            pltpu.sync_copy(ea_hbm.at[pl.ds(eb_base + j * C, C)], r0)
            pltpu.sync_copy(r0, acc_sh.at[dstB_v.at[j]], add=True)

        @pl.when(j + 2 < CH_B)
        def _():
            pltpu.make_async_copy(h_hbm.at[srcB_v.at[j + 2]], r0, sem0).start()
        pltpu.make_async_copy(h_hbm.at[srcB_v.at[j + 1]], r1, sem1).wait()
        pltpu.sync_copy(r1, acc_sh.at[dstB_v.at[j + 1]], add=True)

        @pl.when(j + 1 < FULL_B)
        def _():
            pltpu.sync_copy(ea_hbm.at[pl.ds(eb_base + (j + 1) * C, C)], r1)
            pltpu.sync_copy(r1, acc_sh.at[dstB_v.at[j + 1]], add=True)
        return 0
    lax.fori_loop(0, CH_B // 2, pairB, 0)

    pltpu.sync_copy(ea_hbm.at[pl.ds(eb_base + FULL_B * C, REM_B)],
                    r0.at[pl.ds(0, REM_B)])
    pltpu.sync_copy(r0, acc_sh.at[dstB_v.at[FULL_B]], add=True)

    plsc.subcore_barrier()
    pltpu.sync_copy(acc_sh.at[pl.ds(base, RPT)],
                    out_hbm.at[pl.ds(c * N_ACC + base, RPT)])


_full_call = pl.kernel(
    _full_body,
    out_type=jax.ShapeDtypeStruct((NC * N_ACC, H), jnp.float32),
    mesh=_sc_mesh,
    scratch_types=[
        pltpu.VMEM((G, C), jnp.int32),
        pltpu.VMEM((G, C), jnp.int32),
        pltpu.VMEM((CH_B, C), jnp.int32),
        pltpu.VMEM((CH_B, C), jnp.int32),
        pltpu.VMEM((C, H), jnp.float32),
        pltpu.VMEM((C, H), jnp.float32),
        pltpu.VMEM_SHARED((N_ACC, H), jnp.float32),
        pltpu.SemaphoreType.DMA,
        pltpu.SemaphoreType.DMA,
    ],
)


CH_C = 40


def _poly_body(dstC_hbm, v_hbm, out_hbm, dstC_v, v0, v1, acc_sh, sem0, sem1):
    c = lax.axis_index("c")
    s = lax.axis_index("s")
    w = c * NS + s

    pltpu.sync_copy(dstC_hbm.at[w], dstC_v)

    _zero_rows(v0, C, F16)
    base = s * RPT
    for k in range(RPT // C):
        pltpu.sync_copy(v0, acc_sh.at[pl.ds(base + k * C, C)])
    rem = RPT % C
    if rem:
        pltpu.sync_copy(v0.at[pl.ds(0, rem)],
                        acc_sh.at[pl.ds(base + (RPT // C) * C, rem)])
    plsc.subcore_barrier()

    def vch(j):
        return v_hbm.at[pl.ds((w * CH_C + j) * C, C)]

    pltpu.make_async_copy(vch(0), v0, sem0).start()

    def pairC(j2, _):
        j = 2 * j2
        pltpu.make_async_copy(vch(j + 1), v1, sem1).start()
        pltpu.make_async_copy(vch(j), v0, sem0).wait()
        pltpu.sync_copy(v0, acc_sh.at[dstC_v.at[j]], add=True)

        @pl.when(j + 2 < CH_C)
        def _():
            pltpu.make_async_copy(vch(j + 2), v0, sem0).start()
        pltpu.make_async_copy(vch(j + 1), v1, sem1).wait()
        pltpu.sync_copy(v1, acc_sh.at[dstC_v.at[j + 1]], add=True)
        return 0
    lax.fori_loop(0, CH_C // 2, pairC, 0)

    plsc.subcore_barrier()
    pltpu.sync_copy(acc_sh.at[pl.ds(base, RPT)],
                    out_hbm.at[pl.ds(c * N_ACC + base, RPT)])


_poly_call = pl.kernel(
    _poly_body,
    out_type=jax.ShapeDtypeStruct((NC * N_ACC, H), jnp.float32),
    mesh=_sc_mesh,
    scratch_types=[
        pltpu.VMEM((CH_C, C), jnp.int32),
        pltpu.VMEM((C, H), jnp.float32),
        pltpu.VMEM((C, H), jnp.float32),
        pltpu.VMEM_SHARED((N_ACC, H), jnp.float32),
        pltpu.SemaphoreType.DMA,
        pltpu.SemaphoreType.DMA,
    ],
)


_HI = lax.Precision.HIGHEST


def _bn_relu(y):
    m = jnp.mean(y, axis=0, keepdims=True)
    v = jnp.mean((y - m) * (y - m), axis=0, keepdims=True)
    return jnp.maximum((y - m) / jnp.sqrt(v + 1e-5), 0.0)


def _enc_body(x_ref, w_ref, b_ref, o_ref):
    y = jnp.dot(x_ref[...], w_ref[...], preferred_element_type=jnp.float32,
                precision=_HI) + b_ref[...]
    o_ref[...] = _bn_relu(y)


_enc_call = pl.pallas_call(
    _enc_body, out_shape=jax.ShapeDtypeStruct((N, H), jnp.float32))


def _post_conv_body(p_ref, h_ref, w_ref, b_ref, o_ref):
    agg = p_ref[pl.ds(0, N), :] + p_ref[pl.ds(N_ACC, N), :]
    y = jnp.dot(agg, w_ref[...], preferred_element_type=jnp.float32,
                precision=_HI) + b_ref[...]
    o_ref[...] = _bn_relu(y) + h_ref[...]


_post_conv_call = pl.pallas_call(
    _post_conv_body, out_shape=jax.ShapeDtypeStruct((N, H), jnp.float32))


def _jk1_body(h0_ref, h1_ref, lp_ref, wjk_ref, wl_ref, b_ref, o_ref):
    y = jnp.dot(h0_ref[...], wjk_ref[pl.ds(0, H), :],
                preferred_element_type=jnp.float32, precision=_HI)
    y += jnp.dot(h1_ref[...], wjk_ref[pl.ds(H, H), :],
                 preferred_element_type=jnp.float32, precision=_HI)
    wl = jnp.dot(wl_ref[...], wjk_ref[pl.ds(4 * H, H), :],
                 preferred_element_type=jnp.float32, precision=_HI)
    y += jnp.dot(lp_ref[...], wl, preferred_element_type=jnp.float32,
                 precision=_HI)
    o_ref[...] = y + b_ref[...]


_jk1_call = pl.pallas_call(
    _jk1_body, out_shape=jax.ShapeDtypeStruct((N, H), jnp.float32))


def _jk2_body(y_ref, h2_ref, h3_ref, wjk_ref, o_ref):
    y = y_ref[...]
    y += jnp.dot(h2_ref[...], wjk_ref[pl.ds(2 * H, H), :],
                 preferred_element_type=jnp.float32, precision=_HI)
    y += jnp.dot(h3_ref[...], wjk_ref[pl.ds(3 * H, H), :],
                 preferred_element_type=jnp.float32, precision=_HI)
    o_ref[...] = _bn_relu(y)


_jk2_call = pl.pallas_call(
    _jk2_body, out_shape=jax.ShapeDtypeStruct((N, H), jnp.float32))


def _final_body(p_ref, p16_ref, h_ref, wc_ref, wf_ref, bf_ref, wo_ref,
                bo_ref, o_ref):
    agg = p_ref[pl.ds(0, N), :] + p_ref[pl.ds(N_ACC, N), :]
    p16 = p16_ref[pl.ds(0, N), :] + p16_ref[pl.ds(N_ACC, N), :]
    agg += jnp.dot(p16, wc_ref[...], preferred_element_type=jnp.float32,
                   precision=_HI)
    y = jnp.dot(agg, wf_ref[...], preferred_element_type=jnp.float32,
                precision=_HI) + bf_ref[...]
    h5 = _bn_relu(y) + h_ref[...]
    o_ref[...] = jnp.dot(h5, wo_ref[...], preferred_element_type=jnp.float32,
                         precision=_HI) + bo_ref[...]


_final_call = pl.pallas_call(
    _final_body, out_shape=jax.ShapeDtypeStruct((N, OUT), jnp.float32))


def _slab(a, ch, fill):
    cap = NW * ch * C
    pad = cap - a.shape[0]
    if pad:
        a = jnp.concatenate([a, jnp.full((pad,), fill, a.dtype)])
    return a.reshape(NW, ch, C)


def _slab_il(a, ch, fill, c=C):
    cap = NW * ch * c
    pad = cap - a.shape[0]
    if pad:
        a = jnp.concatenate([a, jnp.full((pad,), fill, a.dtype)])
    return a.reshape(ch, NW, c).swapaxes(0, 1)


def _slab_pt(a, ch, fill):
    ept = a.shape[0] // NW
    a = a.reshape(NW, ept)
    pad = ch * C - ept
    a = jnp.concatenate([a, jnp.full((NW, pad), fill, a.dtype)], axis=1)
    return a.reshape(NW, ch, C)


def kernel(x, edge_index, edge_attr, poly1_index, poly1_val, poly2_index,
           poly2_val, all_poly_index, all_poly_val, all_loop_val, full_index,
           W_enc, b_enc, W0, b0, W1, b1, W2, b2, W_loop, W_conn, W_jk, b_jk,
           W_full, b_full, W_out, b_out):
    h = _enc_call(x, W_enc, b_enc.reshape(1, H))

    blocks = [
        (poly1_index, poly1_val, W0, b0),
        (poly2_index, poly2_val, W1, b1),
        (all_poly_index, all_poly_val[:, -1], W2, b2),
    ]
    nh = [h]
    for pidx, pval, W, b in blocks:
        src = _slab_il(pidx[0], CH_CONV, 0, CC).reshape(NW, CH_CONV * CC)
        dst = _slab_il(pidx[1], CH_CONV, TRASH, CC).reshape(NW, CH_CONV * CC)
        val = _slab_il(pval, CH_CONV, 0.0, CC).reshape(NW, CH_CONV * CC)
        part = _conv_call(h, src, dst, val)
        h = _post_conv_call(part, h, W, b.reshape(1, H))
        nh.append(h)

    y01 = _jk1_call(nh[0], nh[1], all_loop_val, W_jk, W_loop,
                    b_jk.reshape(1, H))
    h = _jk2_call(y01, nh[2], nh[3], W_jk)

    srcA = _slab_il(jnp.concatenate([full_index[0], all_poly_index[0]]),
                    CH_A, 0)
    dstA = _slab_il(jnp.concatenate([full_index[1], all_poly_index[1]]),
                    CH_A, TRASH)
    srcB = _slab_pt(edge_index[0], CH_B, 0)
    dstB = _slab_pt(edge_index[1], CH_B, TRASH)
    dstC = _slab(all_poly_index[1], CH_C, TRASH)
    v128 = jnp.pad(all_poly_val, ((0, NW * CH_C * C - E), (0, H - EMB)))

    part16 = _poly_call(dstC, v128)
    part = _full_call(h, srcA, dstA, srcB, dstB, edge_attr)

    wc128 = jnp.pad(W_conn, ((0, H - EMB), (0, 0)))
    out = _final_call(part, part16, h, wc128, W_full, b_full.reshape(1, H),
                      W_out, b_out.reshape(1, OUT))
    return out
